# baseline (speedup 1.0000x reference)
import functools

import jax
import jax.numpy as jnp
from jax import lax
from jax.experimental import pallas as pl
from jax.experimental.pallas import tpu as pltpu
from jax.experimental.pallas import tpu_sc as plsc

_N = 10000
_E = 320000
_D = 128
_G = 64
_H = 64
_NS = 16
_NC = 2
_GH = _G // _NC
_EPT = _E // _NS
_VPT = _EPT // 16
_CH = _GH * _N
_CHPT = _CH // _NS


def _sc_body(src_hbm, dst_hbm, batch_hbm, c_out_hbm,
             e_src, e_dst, flat_v, w_v, batch_t, inv_t,
             sem_s, sem_d, sem_b, sem_a, sem_z, cnt_s, c_s):
    cid = lax.axis_index("c")
    tid = lax.axis_index("s")
    zero16 = jnp.zeros((16,), jnp.float32)
    one16 = jnp.ones((16,), jnp.float32)

    cp_s = pltpu.async_copy(src_hbm.at[pl.ds(tid * _EPT, _EPT)], e_src, sem_s)
    cp_d = pltpu.async_copy(dst_hbm.at[pl.ds(tid * _EPT, _EPT)], e_dst, sem_d)
    cp_b = pltpu.async_copy(batch_hbm, batch_t, sem_b)

    def fill_z(i, _):
        inv_t[pl.ds(i * 16, 16)] = zero16
        return 0
    lax.fori_loop(0, _N // 16, fill_z, 0, unroll=8)

    z1 = pltpu.async_copy(inv_t, c_s.at[pl.ds(tid * _CHPT, _N)], sem_z)
    z2 = pltpu.async_copy(inv_t, c_s.at[pl.ds(tid * _CHPT + _N, _N)], sem_z)

    @pl.when(tid == 0)
    def _():
        pltpu.sync_copy(inv_t, cnt_s)

    def fill_w1(i, _):
        w_v[pl.ds(i * 16, 16)] = one16
        return 0
    lax.fori_loop(0, _VPT, fill_w1, 0, unroll=8)

    z1.wait()
    z2.wait()
    cp_d.wait()
    plsc.subcore_barrier()

    st_a = pltpu.async_copy(w_v, cnt_s.at[e_dst], sem_a, add=True)

    gbase = cid * _GH
    cp_b.wait()

    def mktab(i, _):
        sl = pl.ds(i * 16, 16)
        gl = batch_t[sl] - gbase
        ok = (gl >= 0) & (gl < _GH)
        batch_t[sl] = jnp.where(ok, gl * _N, _CH)
        return 0
    lax.fori_loop(0, _N // 16, mktab, 0, unroll=8)

    cp_s.wait()

    def flats(i, _):
        sl = pl.ds(i * 16, 16)
        flat_v[sl] = plsc.load_gather(batch_t, [e_dst[sl]]) + e_src[sl]
        return 0
    lax.fori_loop(0, _VPT, flats, 0, unroll=8)

    st_a.wait()
    plsc.subcore_barrier()

    pltpu.sync_copy(cnt_s, inv_t)

    def invert(i, _):
        v = inv_t[pl.ds(i * 16, 16)]
        inv_t[pl.ds(i * 16, 16)] = 1.0 / jnp.maximum(v, 1.0)
        return 0
    lax.fori_loop(0, _N // 16, invert, 0, unroll=8)

    def weights(i, _):
        sl = pl.ds(i * 16, 16)
        w_v[sl] = plsc.load_gather(inv_t, [e_dst[sl]])
        return 0

    nchunk = 5
    cvregs = _VPT // nchunk
    celems = _EPT // nchunk
    st_b = []
    for k in range(nchunk):
        lax.fori_loop(k * cvregs, (k + 1) * cvregs, weights, 0, unroll=5)
        st_b.append(pltpu.async_copy(
            w_v.at[pl.ds(k * celems, celems)],
            c_s.at[flat_v.at[pl.ds(k * celems, celems)]],
            sem_a, add=True))
    for st in st_b:
        st.wait()

    plsc.subcore_barrier()

    half = _CHPT // 2
    obase = cid * _CH + tid * _CHPT
    pltpu.sync_copy(c_s.at[pl.ds(tid * _CHPT, half)], w_v.at[pl.ds(0, half)])
    o1 = pltpu.async_copy(w_v.at[pl.ds(0, half)],
                          c_out_hbm.at[pl.ds(obase, half)], sem_s)
    pltpu.sync_copy(c_s.at[pl.ds(tid * _CHPT + half, half)],
                    w_v.at[pl.ds(half, half)])
    o2 = pltpu.async_copy(w_v.at[pl.ds(half, half)],
                          c_out_hbm.at[pl.ds(obase + half, half)], sem_d)
    o1.wait()
    o2.wait()


@jax.jit
def _sc_build_c(src, dst, batch):
    mesh = plsc.VectorSubcoreMesh(core_axis_name="c", subcore_axis_name="s")
    return pl.kernel(
        _sc_body,
        out_type=jax.ShapeDtypeStruct((_NC * _CH,), jnp.float32),
        mesh=mesh,
        compiler_params=pltpu.CompilerParams(needs_layout_passes=False),
        scratch_types=[
            pltpu.VMEM((_EPT,), jnp.int32),
            pltpu.VMEM((_EPT,), jnp.int32),
            pltpu.VMEM((_EPT,), jnp.int32),
            pltpu.VMEM((_EPT,), jnp.float32),
            pltpu.VMEM((_N,), jnp.int32),
            pltpu.VMEM((_N,), jnp.float32),
            pltpu.SemaphoreType.DMA,
            pltpu.SemaphoreType.DMA,
            pltpu.SemaphoreType.DMA,
            pltpu.SemaphoreType.DMA,
            pltpu.SemaphoreType.DMA,
            pltpu.VMEM_SHARED((_N,), jnp.float32),
            pltpu.VMEM_SHARED((_CH + _N,), jnp.float32),
        ],
        name="sage_edge_pool_sc",
    )(src, dst, batch)


def _tc_body(x_ref, batch_ref, c_ref,
             wl_ref, wr_ref, b0_ref, w1_ref, b1_ref, w2_ref, b2_ref,
             w3_ref, b3_ref, w4_ref, b4_ref, out_ref):
    x = x_ref[...]
    cmat = c_ref[...]
    pooled_mean = jnp.dot(cmat, x, preferred_element_type=jnp.float32)

    b = batch_ref[...].reshape(_N, 1)
    gids = lax.broadcasted_iota(jnp.int32, (1, _G), 1)
    onehot = (b == gids).astype(jnp.float32)
    pooled_x = lax.dot_general(onehot, x, (((0,), (0,)), ((), ())),
                               preferred_element_type=jnp.float32)
    cnt = jnp.sum(onehot, axis=0)

    s = (jnp.dot(pooled_mean, wl_ref[...], preferred_element_type=jnp.float32)
         + jnp.dot(pooled_x, wr_ref[...], preferred_element_type=jnp.float32))
    h = s / jnp.clip(cnt, 1.0)[:, None] + b0_ref[...][None, :]
    h = jnp.maximum(jnp.dot(h, w1_ref[...]) + b1_ref[...][None, :], 0.0)
    h = jnp.maximum(jnp.dot(h, w2_ref[...]) + b2_ref[...][None, :], 0.0)
    h = jnp.maximum(jnp.dot(h, w3_ref[...]) + b3_ref[...][None, :], 0.0)
    out_ref[...] = jnp.dot(h, w4_ref[...]) + b4_ref[...][None, :]


@jax.jit
def _tc_head(x, batch, c_flat, W_l, W_r, b0, W1, b1, W2, b2, W3, b3, W4, b4):
    cmat = c_flat.reshape(_G, _N)
    return pl.pallas_call(
        _tc_body,
        out_shape=jax.ShapeDtypeStruct((_G, 1), jnp.float32),
    )(x, batch, cmat, W_l, W_r, b0, W1, b1, W2, b2, W3, b3, W4, b4)


def kernel(x, edge_index, batch, W_l, W_r, b0, W1, b1, W2, b2, W3, b3, W4, b4):
    src = edge_index[0]
    dst = edge_index[1]
    c_flat = _sc_build_c(src, dst, batch)
    return _tc_head(x, batch, c_flat, W_l, W_r, b0,
                    W1, b1, W2, b2, W3, b3, W4, b4)


# speedup vs baseline: 1.0219x; 1.0219x over previous
import functools

import jax
import jax.numpy as jnp
from jax import lax
from jax.experimental import pallas as pl
from jax.experimental.pallas import tpu as pltpu
from jax.experimental.pallas import tpu_sc as plsc

_N = 10000
_E = 320000
_D = 128
_G = 64
_H = 64
_NS = 16
_NC = 2
_GH = _G // _NC
_EPT = _E // _NS
_VPT = _EPT // 16
_CH = _GH * _N
_CHPT = _CH // _NS


def _sc_body(src_hbm, dst_hbm, batch_hbm, c_out_hbm,
             e_src, e_dst, flat_v, w_v, batch_t, inv_t,
             sem_s, sem_d, sem_b, sem_a, sem_z, cnt_s, c_s):
    cid = lax.axis_index("c")
    tid = lax.axis_index("s")
    zero16 = jnp.zeros((16,), jnp.float32)
    one16 = jnp.ones((16,), jnp.float32)

    cp_s = pltpu.async_copy(src_hbm.at[pl.ds(tid * _EPT, _EPT)], e_src, sem_s)
    cp_d = pltpu.async_copy(dst_hbm.at[pl.ds(tid * _EPT, _EPT)], e_dst, sem_d)
    cp_b = pltpu.async_copy(batch_hbm, batch_t, sem_b)

    def fill_z(i, _):
        inv_t[pl.ds(i * 16, 16)] = zero16
        return 0
    lax.fori_loop(0, _N // 16, fill_z, 0, unroll=8)

    z1 = pltpu.async_copy(inv_t, c_s.at[pl.ds(tid * _CHPT, _N)], sem_z)
    z2 = pltpu.async_copy(inv_t, c_s.at[pl.ds(tid * _CHPT + _N, _N)], sem_z)

    @pl.when(tid == 0)
    def _():
        pltpu.sync_copy(inv_t, cnt_s)

    def fill_w1(i, _):
        w_v[pl.ds(i * 16, 16)] = one16
        return 0
    lax.fori_loop(0, _VPT, fill_w1, 0, unroll=8)

    z1.wait()
    z2.wait()
    cp_d.wait()
    plsc.subcore_barrier()

    st_a = pltpu.async_copy(w_v, cnt_s.at[e_dst], sem_a, add=True)

    gbase = cid * _GH
    cp_b.wait()

    def mktab(i, _):
        sl = pl.ds(i * 16, 16)
        gl = batch_t[sl] - gbase
        ok = (gl >= 0) & (gl < _GH)
        batch_t[sl] = jnp.where(ok, gl * _N, _CH)
        return 0
    lax.fori_loop(0, _N // 16, mktab, 0, unroll=8)

    cp_s.wait()

    def flats(i, _):
        sl = pl.ds(i * 16, 16)
        flat_v[sl] = plsc.load_gather(batch_t, [e_dst[sl]]) + e_src[sl]
        return 0
    lax.fori_loop(0, _VPT, flats, 0, unroll=8)

    st_a.wait()
    plsc.subcore_barrier()

    pltpu.sync_copy(cnt_s, inv_t)

    def invert(i, _):
        v = inv_t[pl.ds(i * 16, 16)]
        inv_t[pl.ds(i * 16, 16)] = 1.0 / jnp.maximum(v, 1.0)
        return 0
    lax.fori_loop(0, _N // 16, invert, 0, unroll=8)

    def weights(i, _):
        sl = pl.ds(i * 16, 16)
        w_v[sl] = plsc.load_gather(inv_t, [e_dst[sl]])
        return 0

    nchunk = 5
    cvregs = _VPT // nchunk
    celems = _EPT // nchunk
    st_b = []
    for k in range(nchunk):
        lax.fori_loop(k * cvregs, (k + 1) * cvregs, weights, 0, unroll=5)
        st_b.append(pltpu.async_copy(
            w_v.at[pl.ds(k * celems, celems)],
            c_s.at[flat_v.at[pl.ds(k * celems, celems)]],
            sem_a, add=True))
    for st in st_b:
        st.wait()

    plsc.subcore_barrier()

    half = _CHPT // 2
    obase = cid * _CH + tid * _CHPT
    pltpu.sync_copy(c_s.at[pl.ds(tid * _CHPT, half)], w_v.at[pl.ds(0, half)])
    o1 = pltpu.async_copy(w_v.at[pl.ds(0, half)],
                          c_out_hbm.at[pl.ds(obase, half)], sem_s)
    pltpu.sync_copy(c_s.at[pl.ds(tid * _CHPT + half, half)],
                    w_v.at[pl.ds(half, half)])
    o2 = pltpu.async_copy(w_v.at[pl.ds(half, half)],
                          c_out_hbm.at[pl.ds(obase + half, half)], sem_d)
    o1.wait()
    o2.wait()


@jax.jit
def _sc_build_c(src, dst, batch):
    mesh = plsc.VectorSubcoreMesh(core_axis_name="c", subcore_axis_name="s")
    return pl.kernel(
        _sc_body,
        out_type=jax.ShapeDtypeStruct((_NC * _CH,), jnp.float32),
        mesh=mesh,
        compiler_params=pltpu.CompilerParams(needs_layout_passes=False),
        scratch_types=[
            pltpu.VMEM((_EPT,), jnp.int32),
            pltpu.VMEM((_EPT,), jnp.int32),
            pltpu.VMEM((_EPT,), jnp.int32),
            pltpu.VMEM((_EPT,), jnp.float32),
            pltpu.VMEM((_N,), jnp.int32),
            pltpu.VMEM((_N,), jnp.float32),
            pltpu.SemaphoreType.DMA,
            pltpu.SemaphoreType.DMA,
            pltpu.SemaphoreType.DMA,
            pltpu.SemaphoreType.DMA,
            pltpu.SemaphoreType.DMA,
            pltpu.VMEM_SHARED((_N,), jnp.float32),
            pltpu.VMEM_SHARED((_CH + _N,), jnp.float32),
        ],
        name="sage_edge_pool_sc",
    )(src, dst, batch)


def _tc_pre_body(x_ref, batch_ref, wl_ref, wr_ref,
                 y_ref, s2_ref, cnt_ref):
    x = x_ref[...]
    y_ref[...] = jnp.dot(x, wl_ref[...], preferred_element_type=jnp.float32)

    b = batch_ref[...].reshape(_N, 1)
    gids = lax.broadcasted_iota(jnp.int32, (1, _G), 1)
    onehot = (b == gids).astype(jnp.float32)
    pooled_x = lax.dot_general(onehot, x, (((0,), (0,)), ((), ())),
                               preferred_element_type=jnp.float32)
    s2_ref[...] = jnp.dot(pooled_x, wr_ref[...],
                          preferred_element_type=jnp.float32)
    cnt_ref[...] = jnp.sum(onehot, axis=0).reshape(_G, 1)


@jax.jit
def _tc_pre(x, batch, W_l, W_r):
    return pl.pallas_call(
        _tc_pre_body,
        out_shape=(
            jax.ShapeDtypeStruct((_N, _H), jnp.float32),
            jax.ShapeDtypeStruct((_G, _H), jnp.float32),
            jax.ShapeDtypeStruct((_G, 1), jnp.float32),
        ),
    )(x, batch, W_l, W_r)


def _tc_post_body(c_ref, y_ref, s2_ref, cnt_ref,
                  b0_ref, w1_ref, b1_ref, w2_ref, b2_ref,
                  w3_ref, b3_ref, w4_ref, b4_ref, out_ref):
    cmat = c_ref[...]
    y = y_ref[...]
    s = jnp.dot(cmat, y, preferred_element_type=jnp.float32) + s2_ref[...]
    h = s / jnp.clip(cnt_ref[...], 1.0) + b0_ref[...][None, :]
    h = jnp.maximum(jnp.dot(h, w1_ref[...]) + b1_ref[...][None, :], 0.0)
    h = jnp.maximum(jnp.dot(h, w2_ref[...]) + b2_ref[...][None, :], 0.0)
    h = jnp.maximum(jnp.dot(h, w3_ref[...]) + b3_ref[...][None, :], 0.0)
    out_ref[...] = jnp.dot(h, w4_ref[...]) + b4_ref[...][None, :]


@jax.jit
def _tc_post(c_flat, y, s2, cnt, b0, W1, b1, W2, b2, W3, b3, W4, b4):
    cmat = c_flat.reshape(_G, _N)
    return pl.pallas_call(
        _tc_post_body,
        out_shape=jax.ShapeDtypeStruct((_G, 1), jnp.float32),
    )(cmat, y, s2, cnt, b0, W1, b1, W2, b2, W3, b3, W4, b4)


def kernel(x, edge_index, batch, W_l, W_r, b0, W1, b1, W2, b2, W3, b3, W4, b4):
    src = edge_index[0]
    dst = edge_index[1]
    c_flat = _sc_build_c(src, dst, batch)
    y, s2, cnt = _tc_pre(x, batch, W_l, W_r)
    return _tc_post(c_flat, y, s2, cnt, b0,
                    W1, b1, W2, b2, W3, b3, W4, b4)
